# baseline (speedup 1.0000x reference)
import functools

import jax
import jax.numpy as jnp
from jax import lax
from jax.experimental import pallas as pl
from jax.experimental.pallas import tpu as pltpu
from jax.experimental.pallas import tpu_sc as plsc

N_NODES = 10000
N_EDGES = 320000
D = 128

NC = 2
NS = 16
NW = NC * NS
EPT = N_EDGES // NW
CH = 80
NSTEP = EPT // CH
N_PAD = CH * D
RPT = N_PAD // NS
DEPTH = 3


def _sc_body(compute_deg, x_hbm, src_hbm, dst_hbm, *refs):
    if compute_deg:
        acc_hbm, deg_hbm = refs[0], refs[1]
        rest = refs[2:]
    else:
        acc_hbm = refs[0]
        rest = refs[1:]
    acc_sh = rest[0]
    rows = rest[1:1 + DEPTH]
    sall = rest[1 + DEPTH]
    didx = rest[2 + DEPTH:2 + 2 * DEPTH]
    gsem = rest[2 + 2 * DEPTH:2 + 3 * DEPTH]
    dsem = rest[2 + 3 * DEPTH:2 + 4 * DEPTH]

    c = lax.axis_index("c")
    s = lax.axis_index("s")
    wid = s * NC + c
    ebase = wid * EPT
    rbase = s * RPT

    def _zero_buf(buf):
        @pl.loop(0, CH)
        def _z(i):
            for j in range(D // 16):
                buf[i, pl.ds(j * 16, 16)] = jnp.zeros((16,), jnp.float32)

    if compute_deg:
        _zero_buf(rows[0])
        pltpu.sync_copy(dst_hbm.at[pl.ds(ebase, EPT)], sall)
        ones = jnp.ones((16,), jnp.float32)

        @pl.loop(0, EPT // 16)
        def _deg(i):
            idx = sall[pl.ds(i * 16, 16)]
            plsc.addupdate_scatter(
                rows[0],
                [lax.shift_right_logical(idx, 7), lax.bitwise_and(idx, 127)],
                ones)

        pltpu.sync_copy(rows[0], deg_hbm.at[wid])

    for b in range(DEPTH):
        _zero_buf(rows[b])

    @pl.loop(0, RPT // CH)
    def _zero_acc(k):
        pltpu.sync_copy(rows[0], acc_sh.at[pl.ds(rbase + k * CH, CH)])

    pltpu.sync_copy(src_hbm.at[pl.ds(ebase, EPT)], sall)

    plsc.subcore_barrier()

    def _start(i, t):
        pltpu.async_copy(dst_hbm.at[pl.ds(ebase + i * CH, CH)],
                         didx[t], dsem[t])
        pltpu.async_copy(x_hbm.at[sall.at[pl.ds(i * CH, CH)]],
                         rows[t], gsem[t])

    def _finish(t):
        pltpu.make_async_copy(dst_hbm.at[pl.ds(0, CH)], didx[t],
                              dsem[t]).wait()
        pltpu.make_async_copy(x_hbm.at[pl.ds(0, CH)], rows[t],
                              gsem[t]).wait()
        pltpu.sync_copy(rows[t], acc_sh.at[didx[t]], add=True)

    for t in range(DEPTH):
        _start(t, t)

    NFULL = (NSTEP - DEPTH) // DEPTH

    @pl.loop(0, NFULL)
    def _edges(j):
        for t in range(DEPTH):
            i = DEPTH * j + t
            _finish(t)
            _start(i + DEPTH, t)

    for i in range(DEPTH * NFULL, NSTEP):
        t = i % DEPTH
        _finish(t)
        if i + DEPTH < NSTEP:
            _start(i + DEPTH, t)

    plsc.subcore_barrier()

    pltpu.sync_copy(acc_sh.at[pl.ds(rbase, RPT)],
                    acc_hbm.at[pl.ds(c * N_PAD + rbase, RPT)])


@functools.lru_cache(maxsize=None)
def _make_sc(compute_deg):
    mesh = plsc.VectorSubcoreMesh(core_axis_name="c", subcore_axis_name="s",
                                  num_cores=NC, num_subcores=NS)
    out_type = [jax.ShapeDtypeStruct((NC * N_PAD, D), jnp.float32)]
    if compute_deg:
        out_type.append(jax.ShapeDtypeStruct((NW, CH, D), jnp.float32))
    scratch = (
        [pltpu.VMEM_SHARED((N_PAD, D), jnp.float32)]
        + [pltpu.VMEM((CH, D), jnp.float32)] * DEPTH
        + [pltpu.VMEM((EPT,), jnp.int32)]
        + [pltpu.VMEM((CH,), jnp.int32)] * DEPTH
        + [pltpu.SemaphoreType.DMA] * (2 * DEPTH)
    )
    return pl.kernel(functools.partial(_sc_body, compute_deg),
                     out_type=tuple(out_type), mesh=mesh,
                     scratch_types=tuple(scratch),
                     compiler_params=pltpu.CompilerParams(
                         needs_layout_passes=False))


ROWS_BLK = 2560
GRID = (N_PAD // ROWS_BLK,)


def _mean(acc_ref, deg_ref):
    deg = jnp.sum(deg_ref[...], axis=0)
    inv = 1.0 / jnp.maximum(deg, 1.0)
    return (acc_ref[0] + acc_ref[1]) * inv[:, None]


def _tc1_body(acc_ref, deg_ref, x_ref, wl_ref, bl_ref, wr_ref, o_ref):
    mean = _mean(acc_ref, deg_ref)
    h = (jnp.dot(mean, wl_ref[...], preferred_element_type=jnp.float32)
         + bl_ref[...]
         + jnp.dot(x_ref[...], wr_ref[...], preferred_element_type=jnp.float32))
    o_ref[...] = jnp.maximum(h, 0.0)


def _tc2_body(acc_ref, deg_ref, h_ref, wl_ref, bl_ref, wr_ref, wc_ref, bc_ref,
              o_ref):
    mean = _mean(acc_ref, deg_ref)
    h = (jnp.dot(mean, wl_ref[...], preferred_element_type=jnp.float32)
         + bl_ref[...]
         + jnp.dot(h_ref[...], wr_ref[...], preferred_element_type=jnp.float32))
    h = jnp.maximum(h, 0.0)
    z = jnp.dot(h, wc_ref[...], preferred_element_type=jnp.float32) + bc_ref[...]
    m = jnp.max(z, axis=1, keepdims=True)
    e = jnp.exp(z - m)
    o_ref[...] = (z - m) - jnp.log(jnp.sum(e, axis=1, keepdims=True))


_acc_spec = pl.BlockSpec((2, ROWS_BLK, D), lambda i: (0, i, 0))
_deg_spec = pl.BlockSpec((NW, ROWS_BLK), lambda i: (0, i))
_row_spec = pl.BlockSpec((ROWS_BLK, D), lambda i: (i, 0))
_w_spec = pl.BlockSpec((D, D), lambda i: (0, 0))
_b_spec = pl.BlockSpec((1, D), lambda i: (0, 0))

_tc1 = pl.pallas_call(
    _tc1_body, grid=GRID,
    in_specs=[_acc_spec, _deg_spec, _row_spec, _w_spec, _b_spec, _w_spec],
    out_specs=_row_spec,
    out_shape=jax.ShapeDtypeStruct((N_PAD, D), jnp.float32))

_tc2 = pl.pallas_call(
    _tc2_body, grid=GRID,
    in_specs=[_acc_spec, _deg_spec, _row_spec, _w_spec, _b_spec, _w_spec,
              _w_spec, _b_spec],
    out_specs=_row_spec,
    out_shape=jax.ShapeDtypeStruct((N_PAD, D), jnp.float32))


@jax.jit
def kernel(x, edge_index, W1l, b1l, W1r, W2l, b2l, W2r, Wc, bc):
    ei = edge_index.astype(jnp.int32)
    src = ei[0]
    dst = ei[1]
    xp = jnp.pad(x, ((0, N_PAD - N_NODES), (0, 0)))
    acc1, degp = _make_sc(True)(xp, src, dst)
    degp = degp.reshape(NW, N_PAD)
    h1 = _tc1(acc1.reshape(2, N_PAD, D), degp, xp,
              W1l.T, b1l.reshape(1, D), W1r.T)
    acc2, = _make_sc(False)(h1, src, dst)
    out = _tc2(acc2.reshape(2, N_PAD, D), degp, h1,
               W2l.T, b2l.reshape(1, D), W2r.T, Wc.T, bc.reshape(1, D))
    return out[:N_NODES]


# speedup vs baseline: 1.0187x; 1.0187x over previous
import functools

import jax
import jax.numpy as jnp
from jax import lax
from jax.experimental import pallas as pl
from jax.experimental.pallas import tpu as pltpu
from jax.experimental.pallas import tpu_sc as plsc

N_NODES = 10000
N_EDGES = 320000
D = 128

NC = 2
NS = 16
NW = NC * NS
EPT = N_EDGES // NW
CH = 80
NSTEP = EPT // CH
N_PAD = CH * D
RPT = N_PAD // NS
DEPTH = 3


def _sc_body(compute_deg, x_hbm, src_hbm, dst_hbm, *refs):
    if compute_deg:
        acc_hbm, deg_hbm = refs[0], refs[1]
        rest = refs[2:]
    else:
        acc_hbm = refs[0]
        rest = refs[1:]
    acc_sh = rest[0]
    rows = rest[1:1 + DEPTH]
    sall = rest[1 + DEPTH]
    didx = rest[2 + DEPTH:2 + 2 * DEPTH]
    gsem = rest[2 + 2 * DEPTH:2 + 3 * DEPTH]
    dsem = rest[2 + 3 * DEPTH:2 + 4 * DEPTH]

    c = lax.axis_index("c")
    s = lax.axis_index("s")
    wid = s * NC + c
    ebase = wid * EPT
    rbase = s * RPT

    def _zero_buf(buf):
        @pl.loop(0, CH)
        def _z(i):
            for j in range(D // 16):
                buf[i, pl.ds(j * 16, 16)] = jnp.zeros((16,), jnp.float32)

    if compute_deg:
        _zero_buf(rows[0])
        pltpu.sync_copy(dst_hbm.at[pl.ds(ebase, EPT)], sall)
        ones = jnp.ones((16,), jnp.float32)

        @pl.loop(0, EPT // 16)
        def _deg(i):
            idx = sall[pl.ds(i * 16, 16)]
            plsc.addupdate_scatter(
                rows[0],
                [lax.shift_right_logical(idx, 7), lax.bitwise_and(idx, 127)],
                ones)

        pltpu.sync_copy(rows[0], deg_hbm.at[wid])

    for b in range(DEPTH):
        _zero_buf(rows[b])

    @pl.loop(0, RPT // CH)
    def _zero_acc(k):
        pltpu.sync_copy(rows[0], acc_sh.at[pl.ds(rbase + k * CH, CH)])

    pltpu.sync_copy(src_hbm.at[pl.ds(ebase, EPT)], sall)

    plsc.subcore_barrier()

    def _start(i, t):
        pltpu.async_copy(dst_hbm.at[pl.ds(ebase + i * CH, CH)],
                         didx[t], dsem[t])
        pltpu.async_copy(x_hbm.at[sall.at[pl.ds(i * CH, CH)]],
                         rows[t], gsem[t])

    def _finish(t):
        pltpu.make_async_copy(dst_hbm.at[pl.ds(0, CH)], didx[t],
                              dsem[t]).wait()
        pltpu.make_async_copy(x_hbm.at[pl.ds(0, CH)], rows[t],
                              gsem[t]).wait()
        pltpu.sync_copy(rows[t], acc_sh.at[didx[t]], add=True)

    for t in range(DEPTH):
        _start(t, t)

    NFULL = (NSTEP - DEPTH) // DEPTH

    @pl.loop(0, NFULL)
    def _edges(j):
        for t in range(DEPTH):
            i = DEPTH * j + t
            _finish(t)
            _start(i + DEPTH, t)

    for i in range(DEPTH * NFULL, NSTEP):
        t = i % DEPTH
        _finish(t)
        if i + DEPTH < NSTEP:
            _start(i + DEPTH, t)

    plsc.subcore_barrier()

    pltpu.sync_copy(acc_sh.at[pl.ds(rbase, RPT)],
                    acc_hbm.at[pl.ds(c * N_PAD + rbase, RPT)])


@functools.lru_cache(maxsize=None)
def _make_sc(compute_deg):
    mesh = plsc.VectorSubcoreMesh(core_axis_name="c", subcore_axis_name="s",
                                  num_cores=NC, num_subcores=NS)
    out_type = [jax.ShapeDtypeStruct((NC * N_PAD, D), jnp.float32)]
    if compute_deg:
        out_type.append(jax.ShapeDtypeStruct((NW, CH, D), jnp.float32))
    scratch = (
        [pltpu.VMEM_SHARED((N_PAD, D), jnp.float32)]
        + [pltpu.VMEM((CH, D), jnp.float32)] * DEPTH
        + [pltpu.VMEM((EPT,), jnp.int32)]
        + [pltpu.VMEM((CH,), jnp.int32)] * DEPTH
        + [pltpu.SemaphoreType.DMA] * (2 * DEPTH)
    )
    return pl.kernel(functools.partial(_sc_body, compute_deg),
                     out_type=tuple(out_type), mesh=mesh,
                     scratch_types=tuple(scratch),
                     compiler_params=pltpu.CompilerParams(
                         needs_layout_passes=False))


GRID = (1,)


def _mean(acc_ref, deg_ref):
    deg = jnp.sum(deg_ref[...], axis=0)[:N_NODES]
    inv = 1.0 / jnp.maximum(deg, 1.0)
    s = acc_ref[0, :N_NODES] + acc_ref[1, :N_NODES]
    return s * inv[:, None]


def _tc1_body(acc_ref, deg_ref, x_ref, wl_ref, bl_ref, wr_ref, o_ref):
    mean = _mean(acc_ref, deg_ref)
    h = (jnp.dot(mean, wl_ref[...], preferred_element_type=jnp.float32)
         + bl_ref[...]
         + jnp.dot(x_ref[...], wr_ref[...], preferred_element_type=jnp.float32))
    o_ref[...] = jnp.maximum(h, 0.0)


def _tc2_body(acc_ref, deg_ref, h_ref, wl_ref, bl_ref, wr_ref, wc_ref, bc_ref,
              o_ref):
    mean = _mean(acc_ref, deg_ref)
    h = (jnp.dot(mean, wl_ref[...], preferred_element_type=jnp.float32)
         + bl_ref[...]
         + jnp.dot(h_ref[...], wr_ref[...], preferred_element_type=jnp.float32))
    h = jnp.maximum(h, 0.0)
    z = jnp.dot(h, wc_ref[...], preferred_element_type=jnp.float32) + bc_ref[...]
    m = jnp.max(z, axis=1, keepdims=True)
    e = jnp.exp(z - m)
    o_ref[...] = (z - m) - jnp.log(jnp.sum(e, axis=1, keepdims=True))


_acc_spec = pl.BlockSpec((2, N_PAD, D), lambda i: (0, 0, 0))
_deg_spec = pl.BlockSpec((NW, N_PAD), lambda i: (0, 0))
_row_spec = pl.BlockSpec((N_NODES, D), lambda i: (0, 0))
_w_spec = pl.BlockSpec((D, D), lambda i: (0, 0))
_b_spec = pl.BlockSpec((1, D), lambda i: (0, 0))

_tc1 = pl.pallas_call(
    _tc1_body, grid=GRID,
    in_specs=[_acc_spec, _deg_spec, _row_spec, _w_spec, _b_spec, _w_spec],
    out_specs=_row_spec,
    out_shape=jax.ShapeDtypeStruct((N_NODES, D), jnp.float32))

_tc2 = pl.pallas_call(
    _tc2_body, grid=GRID,
    in_specs=[_acc_spec, _deg_spec, _row_spec, _w_spec, _b_spec, _w_spec,
              _w_spec, _b_spec],
    out_specs=_row_spec,
    out_shape=jax.ShapeDtypeStruct((N_NODES, D), jnp.float32))


@jax.jit
def kernel(x, edge_index, W1l, b1l, W1r, W2l, b2l, W2r, Wc, bc):
    ei = edge_index.astype(jnp.int32)
    src = ei[0]
    dst = ei[1]
    acc1, degp = _make_sc(True)(x, src, dst)
    degp = degp.reshape(NW, N_PAD)
    h1 = _tc1(acc1.reshape(2, N_PAD, D), degp, x,
              W1l.T, b1l.reshape(1, D), W1r.T)
    acc2, = _make_sc(False)(h1, src, dst)
    out = _tc2(acc2.reshape(2, N_PAD, D), degp, h1,
               W2l.T, b2l.reshape(1, D), W2r.T, Wc.T, bc.reshape(1, D))
    return out


# speedup vs baseline: 1.0331x; 1.0142x over previous
import functools

import jax
import jax.numpy as jnp
from jax import lax
from jax.experimental import pallas as pl
from jax.experimental.pallas import tpu as pltpu
from jax.experimental.pallas import tpu_sc as plsc

N_NODES = 10000
N_EDGES = 320000
D = 128

NC = 2
NS = 16
NW = NC * NS
EPT = N_EDGES // NW
CH = 80
NSTEP = EPT // CH
N_PAD = CH * D
RPT = N_PAD // NS
DEPTH = 3


def _sc_body(compute_deg, x_hbm, src_hbm, dst_hbm, *refs):
    acc_hbm = refs[0]
    rest = refs[1:]
    acc_sh = rest[0]
    rows = rest[1:1 + DEPTH]
    sall = rest[1 + DEPTH]
    didx = rest[2 + DEPTH:2 + 2 * DEPTH]
    gsem = rest[2 + 2 * DEPTH:2 + 3 * DEPTH]
    dsem = rest[2 + 3 * DEPTH:2 + 4 * DEPTH]

    c = lax.axis_index("c")
    s = lax.axis_index("s")
    wid = s * NC + c
    ebase = wid * EPT
    rbase = s * RPT

    def _zero_buf(buf):
        @pl.loop(0, CH)
        def _z(i):
            for j in range(D // 16):
                buf[i, pl.ds(j * 16, 16)] = jnp.zeros((16,), jnp.float32)

    if compute_deg:
        _zero_buf(rows[0])
        pltpu.sync_copy(dst_hbm.at[pl.ds(ebase, EPT)], sall)
        ones = jnp.ones((16,), jnp.float32)

        @pl.loop(0, EPT // 16)
        def _deg(i):
            idx = sall[pl.ds(i * 16, 16)]
            plsc.addupdate_scatter(
                rows[0],
                [lax.shift_right_logical(idx, 7), lax.bitwise_and(idx, 127)],
                ones)

        @pl.loop(0, CH // 16)
        def _ident(k):
            didx[0][pl.ds(k * 16, 16)] = (
                lax.broadcasted_iota(jnp.int32, (16,), 0)
                + (N_NODES + k * 16))

    _zero_buf(rows[1])

    @pl.loop(0, RPT // CH)
    def _zero_acc(k):
        pltpu.sync_copy(rows[1], acc_sh.at[pl.ds(rbase + k * CH, CH)])

    pltpu.sync_copy(src_hbm.at[pl.ds(ebase, EPT)], sall)

    plsc.subcore_barrier()

    if compute_deg:
        pltpu.sync_copy(rows[0], acc_sh.at[didx[0]], add=True)

    def _start(i, t):
        pltpu.async_copy(dst_hbm.at[pl.ds(ebase + i * CH, CH)],
                         didx[t], dsem[t])
        pltpu.async_copy(x_hbm.at[sall.at[pl.ds(i * CH, CH)]],
                         rows[t], gsem[t])

    def _finish(t):
        pltpu.make_async_copy(dst_hbm.at[pl.ds(0, CH)], didx[t],
                              dsem[t]).wait()
        pltpu.make_async_copy(x_hbm.at[pl.ds(0, CH)], rows[t],
                              gsem[t]).wait()
        pltpu.sync_copy(rows[t], acc_sh.at[didx[t]], add=True)

    for t in range(DEPTH):
        _start(t, t)

    NFULL = (NSTEP - DEPTH) // DEPTH

    @pl.loop(0, NFULL)
    def _edges(j):
        for t in range(DEPTH):
            i = DEPTH * j + t
            _finish(t)
            _start(i + DEPTH, t)

    for i in range(DEPTH * NFULL, NSTEP):
        t = i % DEPTH
        _finish(t)
        if i + DEPTH < NSTEP:
            _start(i + DEPTH, t)

    plsc.subcore_barrier()

    pltpu.sync_copy(acc_sh.at[pl.ds(rbase, RPT)],
                    acc_hbm.at[pl.ds(c * N_PAD + rbase, RPT)])


@functools.lru_cache(maxsize=None)
def _make_sc(compute_deg):
    mesh = plsc.VectorSubcoreMesh(core_axis_name="c", subcore_axis_name="s",
                                  num_cores=NC, num_subcores=NS)
    out_type = [jax.ShapeDtypeStruct((NC * N_PAD, D), jnp.float32)]
    scratch = (
        [pltpu.VMEM_SHARED((N_PAD, D), jnp.float32)]
        + [pltpu.VMEM((CH, D), jnp.float32)] * DEPTH
        + [pltpu.VMEM((EPT,), jnp.int32)]
        + [pltpu.VMEM((CH,), jnp.int32)] * DEPTH
        + [pltpu.SemaphoreType.DMA] * (2 * DEPTH)
    )
    return pl.kernel(functools.partial(_sc_body, compute_deg),
                     out_type=tuple(out_type), mesh=mesh,
                     scratch_types=tuple(scratch),
                     compiler_params=pltpu.CompilerParams(
                         needs_layout_passes=False))


GRID = (1,)


def _mean(acc_ref, deg_ref):
    deg = jnp.sum(deg_ref[...], axis=0)[:N_NODES]
    inv = 1.0 / jnp.maximum(deg, 1.0)
    s = acc_ref[0, :N_NODES] + acc_ref[1, :N_NODES]
    return s * inv[:, None]


def _tc1_body(acc_ref, deg_ref, x_ref, wl_ref, bl_ref, wr_ref, o_ref):
    mean = _mean(acc_ref, deg_ref)
    h = (jnp.dot(mean, wl_ref[...], preferred_element_type=jnp.float32)
         + bl_ref[...]
         + jnp.dot(x_ref[...], wr_ref[...], preferred_element_type=jnp.float32))
    o_ref[...] = jnp.maximum(h, 0.0)


def _tc2_body(acc_ref, deg_ref, h_ref, wl_ref, bl_ref, wr_ref, wc_ref, bc_ref,
              o_ref):
    mean = _mean(acc_ref, deg_ref)
    h = (jnp.dot(mean, wl_ref[...], preferred_element_type=jnp.float32)
         + bl_ref[...]
         + jnp.dot(h_ref[...], wr_ref[...], preferred_element_type=jnp.float32))
    h = jnp.maximum(h, 0.0)
    z = jnp.dot(h, wc_ref[...], preferred_element_type=jnp.float32) + bc_ref[...]
    m = jnp.max(z, axis=1, keepdims=True)
    e = jnp.exp(z - m)
    o_ref[...] = (z - m) - jnp.log(jnp.sum(e, axis=1, keepdims=True))


_acc_spec = pl.BlockSpec((2, N_PAD, D), lambda i: (0, 0, 0))
_deg_spec = pl.BlockSpec((2, N_PAD), lambda i: (0, 0))
_row_spec = pl.BlockSpec((N_NODES, D), lambda i: (0, 0))
_w_spec = pl.BlockSpec((D, D), lambda i: (0, 0))
_b_spec = pl.BlockSpec((1, D), lambda i: (0, 0))

_tc1 = pl.pallas_call(
    _tc1_body, grid=GRID,
    in_specs=[_acc_spec, _deg_spec, _row_spec, _w_spec, _b_spec, _w_spec],
    out_specs=_row_spec,
    out_shape=jax.ShapeDtypeStruct((N_NODES, D), jnp.float32))

_tc2 = pl.pallas_call(
    _tc2_body, grid=GRID,
    in_specs=[_acc_spec, _deg_spec, _row_spec, _w_spec, _b_spec, _w_spec,
              _w_spec, _b_spec],
    out_specs=_row_spec,
    out_shape=jax.ShapeDtypeStruct((N_NODES, D), jnp.float32))


@jax.jit
def kernel(x, edge_index, W1l, b1l, W1r, W2l, b2l, W2r, Wc, bc):
    ei = edge_index.astype(jnp.int32)
    src = ei[0]
    dst = ei[1]
    acc1, = _make_sc(True)(x, src, dst)
    acc1 = acc1.reshape(2, N_PAD, D)
    degp = acc1[:, N_NODES:N_NODES + CH, :].reshape(2, N_PAD)
    h1 = _tc1(acc1, degp, x, W1l.T, b1l.reshape(1, D), W1r.T)
    acc2, = _make_sc(False)(h1, src, dst)
    out = _tc2(acc2.reshape(2, N_PAD, D), degp, h1,
               W2l.T, b2l.reshape(1, D), W2r.T, Wc.T, bc.reshape(1, D))
    return out


# speedup vs baseline: 1.0381x; 1.0048x over previous
import functools

import jax
import jax.numpy as jnp
from jax import lax
from jax.experimental import pallas as pl
from jax.experimental.pallas import tpu as pltpu
from jax.experimental.pallas import tpu_sc as plsc

N_NODES = 10000
N_EDGES = 320000
D = 128

NC = 2
NS = 16
NW = NC * NS
EPT = N_EDGES // NW
CH = 80
NSTEP = EPT // CH
N_PAD = CH * D
RPT = N_PAD // NS
DEPTH = 3


def _sc_body(compute_deg, x_hbm, src_hbm, dst_hbm, *refs):
    acc_hbm = refs[0]
    rest = refs[1:]
    acc_sh = rest[0]
    rows = rest[1:1 + DEPTH]
    sall = rest[1 + DEPTH]
    didx = rest[2 + DEPTH:2 + 2 * DEPTH]
    zbuf = rest[2 + 2 * DEPTH]
    gsem = rest[3 + 2 * DEPTH:3 + 3 * DEPTH]
    dsem = rest[3 + 3 * DEPTH:3 + 4 * DEPTH]

    c = lax.axis_index("c")
    s = lax.axis_index("s")
    wid = s * NC + c
    ebase = wid * EPT
    rbase = s * RPT

    def _zero_buf(buf):
        @pl.loop(0, CH)
        def _z(i):
            for j in range(D // 16):
                buf[i, pl.ds(j * 16, 16)] = jnp.zeros((16,), jnp.float32)

    if compute_deg:
        _zero_buf(rows[0])
        pltpu.sync_copy(dst_hbm.at[pl.ds(ebase, EPT)], sall)
        ones = jnp.ones((16,), jnp.float32)

        @pl.loop(0, EPT // 16)
        def _deg(i):
            idx = sall[pl.ds(i * 16, 16)]
            plsc.addupdate_scatter(
                rows[0],
                [lax.shift_right_logical(idx, 7), lax.bitwise_and(idx, 127)],
                ones)

        @pl.loop(0, CH // 16)
        def _ident(k):
            didx[0][pl.ds(k * 16, 16)] = (
                lax.broadcasted_iota(jnp.int32, (16,), 0)
                + (N_NODES + k * 16))

    def _start(i, t):
        pltpu.async_copy(dst_hbm.at[pl.ds(ebase + i * CH, CH)],
                         didx[t], dsem[t])
        pltpu.async_copy(x_hbm.at[sall.at[pl.ds(i * CH, CH)]],
                         rows[t], gsem[t])

    def _finish(t):
        pltpu.make_async_copy(dst_hbm.at[pl.ds(0, CH)], didx[t],
                              dsem[t]).wait()
        pltpu.make_async_copy(x_hbm.at[pl.ds(0, CH)], rows[t],
                              gsem[t]).wait()
        pltpu.sync_copy(rows[t], acc_sh.at[didx[t]], add=True)

    pltpu.sync_copy(src_hbm.at[pl.ds(ebase, EPT)], sall)
    if not compute_deg:
        for t in range(DEPTH):
            _start(t, t)

    @pl.loop(0, CH // 2)
    def _zero_zbuf(i):
        for j in range(D // 16):
            zbuf[i, pl.ds(j * 16, 16)] = jnp.zeros((16,), jnp.float32)

    @pl.loop(0, RPT // (CH // 2))
    def _zero_acc(k):
        pltpu.sync_copy(zbuf, acc_sh.at[pl.ds(rbase + k * (CH // 2), CH // 2)])

    plsc.subcore_barrier()

    if compute_deg:
        pltpu.sync_copy(rows[0], acc_sh.at[didx[0]], add=True)
        for t in range(DEPTH):
            _start(t, t)

    NFULL = (NSTEP - DEPTH) // DEPTH

    @pl.loop(0, NFULL)
    def _edges(j):
        for t in range(DEPTH):
            i = DEPTH * j + t
            _finish(t)
            _start(i + DEPTH, t)

    for i in range(DEPTH * NFULL, NSTEP):
        t = i % DEPTH
        _finish(t)
        if i + DEPTH < NSTEP:
            _start(i + DEPTH, t)

    plsc.subcore_barrier()

    pltpu.sync_copy(acc_sh.at[pl.ds(rbase, RPT)],
                    acc_hbm.at[pl.ds(c * N_PAD + rbase, RPT)])


@functools.lru_cache(maxsize=None)
def _make_sc(compute_deg):
    mesh = plsc.VectorSubcoreMesh(core_axis_name="c", subcore_axis_name="s",
                                  num_cores=NC, num_subcores=NS)
    out_type = [jax.ShapeDtypeStruct((NC * N_PAD, D), jnp.float32)]
    scratch = (
        [pltpu.VMEM_SHARED((N_PAD, D), jnp.float32)]
        + [pltpu.VMEM((CH, D), jnp.float32)] * DEPTH
        + [pltpu.VMEM((EPT,), jnp.int32)]
        + [pltpu.VMEM((CH,), jnp.int32)] * DEPTH
        + [pltpu.VMEM((CH // 2, D), jnp.float32)]
        + [pltpu.SemaphoreType.DMA] * (2 * DEPTH)
    )
    return pl.kernel(functools.partial(_sc_body, compute_deg),
                     out_type=tuple(out_type), mesh=mesh,
                     scratch_types=tuple(scratch),
                     compiler_params=pltpu.CompilerParams(
                         needs_layout_passes=False))


GRID = (1,)


def _mean(acc_ref, deg_ref):
    deg = jnp.sum(deg_ref[...], axis=0)[:N_NODES]
    inv = 1.0 / jnp.maximum(deg, 1.0)
    s = acc_ref[0, :N_NODES] + acc_ref[1, :N_NODES]
    return s * inv[:, None]


def _tc1_body(acc_ref, deg_ref, x_ref, wl_ref, bl_ref, wr_ref, o_ref):
    mean = _mean(acc_ref, deg_ref)
    h = (jnp.dot(mean, wl_ref[...], preferred_element_type=jnp.float32)
         + bl_ref[...]
         + jnp.dot(x_ref[...], wr_ref[...], preferred_element_type=jnp.float32))
    o_ref[...] = jnp.maximum(h, 0.0)


def _tc2_body(acc_ref, deg_ref, h_ref, wl_ref, bl_ref, wr_ref, wc_ref, bc_ref,
              o_ref):
    mean = _mean(acc_ref, deg_ref)
    h = (jnp.dot(mean, wl_ref[...], preferred_element_type=jnp.float32)
         + bl_ref[...]
         + jnp.dot(h_ref[...], wr_ref[...], preferred_element_type=jnp.float32))
    h = jnp.maximum(h, 0.0)
    z = jnp.dot(h, wc_ref[...], preferred_element_type=jnp.float32) + bc_ref[...]
    m = jnp.max(z, axis=1, keepdims=True)
    e = jnp.exp(z - m)
    o_ref[...] = (z - m) - jnp.log(jnp.sum(e, axis=1, keepdims=True))


_acc_spec = pl.BlockSpec((2, N_PAD, D), lambda i: (0, 0, 0))
_deg_spec = pl.BlockSpec((2, N_PAD), lambda i: (0, 0))
_row_spec = pl.BlockSpec((N_NODES, D), lambda i: (0, 0))
_w_spec = pl.BlockSpec((D, D), lambda i: (0, 0))
_b_spec = pl.BlockSpec((1, D), lambda i: (0, 0))

_tc1 = pl.pallas_call(
    _tc1_body, grid=GRID,
    in_specs=[_acc_spec, _deg_spec, _row_spec, _w_spec, _b_spec, _w_spec],
    out_specs=_row_spec,
    out_shape=jax.ShapeDtypeStruct((N_NODES, D), jnp.float32))

_tc2 = pl.pallas_call(
    _tc2_body, grid=GRID,
    in_specs=[_acc_spec, _deg_spec, _row_spec, _w_spec, _b_spec, _w_spec,
              _w_spec, _b_spec],
    out_specs=_row_spec,
    out_shape=jax.ShapeDtypeStruct((N_NODES, D), jnp.float32))


@jax.jit
def kernel(x, edge_index, W1l, b1l, W1r, W2l, b2l, W2r, Wc, bc):
    ei = edge_index.astype(jnp.int32)
    src = ei[0]
    dst = ei[1]
    acc1, = _make_sc(True)(x, src, dst)
    acc1 = acc1.reshape(2, N_PAD, D)
    degp = acc1[:, N_NODES:N_NODES + CH, :].reshape(2, N_PAD)
    h1 = _tc1(acc1, degp, x, W1l.T, b1l.reshape(1, D), W1r.T)
    acc2, = _make_sc(False)(h1, src, dst)
    out = _tc2(acc2.reshape(2, N_PAD, D), degp, h1,
               W2l.T, b2l.reshape(1, D), W2r.T, Wc.T, bc.reshape(1, D))
    return out
